# Initial kernel scaffold; baseline (speedup 1.0000x reference)
import jax
import jax.numpy as jnp
from jax.experimental import pallas as pl


def kernel(X, A, kernel):
    raise NotImplementedError("write your pallas kernel here")


# speedup vs baseline: 2.5798x; 2.5798x over previous
import functools
import math

import jax
import jax.numpy as jnp
from jax import lax
from jax.experimental import pallas as pl
from jax.experimental.pallas import tpu as pltpu
from jax.experimental.pallas import tpu_sc as plsc

N = 10000
F = 512
K = 5000
NW = 32
KPAD = 5024
STRIPE = 160
CH = 8
GB = 4
MININT = -(2**31)


def _score_gate_body(x_ref, p_ref, xg_ref, y_ref):
    x = x_ref[...]
    p = p_ref[...]
    kn = p / jnp.sqrt(jnp.sum(p * p))
    y = jnp.dot(x, kn, preferred_element_type=jnp.float32)
    xg_ref[...] = x * jnp.tanh(y)
    y_ref[...] = y


def _score_gate(x, p):
    return pl.pallas_call(
        _score_gate_body,
        out_shape=[
            jax.ShapeDtypeStruct((N, F), jnp.float32),
            jax.ShapeDtypeStruct((N, 1), jnp.float32),
        ],
    )(x, p)


def _threshold_body(y_ref, meta_ref):
    yv = y_ref[...]
    s = lax.bitcast_convert_type(yv, jnp.int32)
    key = jnp.where(s < 0, s ^ jnp.int32(0x7FFFFFFF), s)

    def bit_body(b, cand):
        bit = 31 - b
        c2 = cand | (jnp.int32(1) << bit)
        cnt = jnp.sum((key >= (c2 ^ jnp.int32(MININT))).astype(jnp.int32))
        return lax.select(cnt >= K, c2, cand)

    cand = lax.fori_loop(0, 32, bit_body, jnp.int32(0))
    ts = cand ^ jnp.int32(MININT)
    cnt_gt = jnp.sum((key > ts).astype(jnp.int32))
    budget = jnp.int32(K) - cnt_gt
    rows = lax.broadcasted_iota(jnp.int32, (8, 128), 0)
    meta_ref[...] = jnp.where(rows == 0, ts, jnp.where(rows == 1, budget, jnp.int32(0)))


def _threshold(y2):
    return pl.pallas_call(
        _threshold_body,
        out_shape=jax.ShapeDtypeStruct((8, 128), jnp.int32),
    )(y2)


def _sc_pool_body(scores_hbm, meta_hbm, a_hbm, xg_hbm, xp_hbm, ap_hbm,
                  scores_v, thr_v, bud_v, idx_v, rowbuf, outbuf, xbuf,
                  sem_a, sem_x):
    cid = lax.axis_index("c")
    sid = lax.axis_index("s")
    w = sid * 2 + cid

    pltpu.sync_copy(scores_hbm, scores_v)
    pltpu.sync_copy(meta_hbm.at[0], thr_v)
    pltpu.sync_copy(meta_hbm.at[1], bud_v)
    thr = thr_v[pl.ds(0, 16)]
    bud = bud_v[pl.ds(0, 16)]
    iota = lax.iota(jnp.int32, 16)

    def init_body(j, carry):
        idx_v[pl.ds(j * 16, 16)] = j * 16 + iota
        return carry

    lax.fori_loop(0, (NW * STRIPE) // 16, init_body, jnp.int32(0))

    one = jnp.broadcast_to(jnp.int32(1), (16,))
    zero = jnp.broadcast_to(jnp.int32(0), (16,))

    def comp_body(j, carry):
        off, tie = carry
        s = scores_v[pl.ds(j * 16, 16)]
        key = jnp.where(s < 0, s ^ jnp.int32(0x7FFFFFFF), s)
        gt = key > thr
        eq = key == thr
        eqi = jnp.where(eq, one, zero)
        exc_eq = plsc.cumsum(eqi) - eqi
        take_eq = jnp.logical_and(eq, (exc_eq + tie) < bud)
        sel = jnp.logical_or(gt, take_eq)
        seli = jnp.where(sel, one, zero)
        pos = jnp.where(sel, off + (plsc.cumsum(seli) - seli),
                        jnp.int32(NW * STRIPE) + iota)
        plsc.store_scatter(idx_v, [pos], j * 16 + iota)
        return (off + jnp.sum(seli), tie + jnp.sum(eqi))

    lax.fori_loop(0, N // 16, comp_body, (jnp.int32(0), jnp.int32(0)))

    base = w * STRIPE
    end = jnp.minimum(base + STRIPE, K)

    def chunk_body(t, carry):
        rowbase = base + t * CH

        @pl.when(rowbase < end)
        def _():
            idx_sl = idx_v.at[pl.ds(rowbase, CH)]
            ha = pltpu.async_copy(a_hbm.at[idx_sl], rowbuf, sem_a)
            hx = pltpu.async_copy(xg_hbm.at[idx_sl], xbuf, sem_x)
            ha.wait()
            hx.wait()
            for g in range(CH // GB):

                def col_body(j, c, g=g):
                    civ = idx_v[pl.ds(j * 16, 16)]
                    for r in range(GB):
                        rv = jnp.broadcast_to(jnp.int32(g * GB + r), (16,))
                        outbuf[pl.ds(r * KPAD + j * 16, 16)] = plsc.load_gather(
                            rowbuf, [rv, civ])
                    return c

                lax.fori_loop(0, KPAD // 16, col_body, jnp.int32(0))
                for r in range(GB):
                    row = rowbase + g * GB + r
                    pltpu.sync_copy(outbuf.at[pl.ds(r * KPAD, K)], ap_hbm.at[row])
            pltpu.sync_copy(xbuf, xp_hbm.at[pl.ds(rowbase, CH)])

        return carry

    lax.fori_loop(0, STRIPE // CH, chunk_body, jnp.int32(0))


@functools.cache
def _sc_pool_build():
    return functools.partial(
        pl.kernel,
        mesh=plsc.VectorSubcoreMesh(core_axis_name="c", subcore_axis_name="s"),
        compiler_params=pltpu.CompilerParams(
            needs_layout_passes=False, use_tc_tiling_on_sc=False),
        out_type=[
            jax.ShapeDtypeStruct((K, F), jnp.float32),
            jax.ShapeDtypeStruct((K, K), jnp.float32),
        ],
        scratch_types=[
            pltpu.VMEM((N,), jnp.int32),
            pltpu.VMEM((128,), jnp.int32),
            pltpu.VMEM((128,), jnp.int32),
            pltpu.VMEM((NW * STRIPE + 16,), jnp.int32),
            pltpu.VMEM((CH, N), jnp.float32),
            pltpu.VMEM((GB * KPAD,), jnp.float32),
            pltpu.VMEM((CH, F), jnp.float32),
            pltpu.SemaphoreType.DMA,
            pltpu.SemaphoreType.DMA,
        ],
    )(_sc_pool_body)


def kernel(X, A, kernel):
    xg, y = _score_gate(X, kernel)
    meta = _threshold(y.reshape(10, 1000))
    y_bits = lax.bitcast_convert_type(y.reshape(N), jnp.int32)
    xp, ap = _sc_pool_build()(y_bits, meta, A, xg)
    return (xp, ap)
